# baseline (speedup 1.0000x reference)
import functools

import numpy as np

import jax
import jax.numpy as jnp
from jax import lax
from jax.experimental import pallas as pl
from jax.experimental.pallas import tpu as pltpu
from jax.experimental.pallas import tpu_sc as plsc

L = 2048
N = L * L
NFA = 8192
NW = 32
APT = NFA // NW
ROWS = L // NW
_NBR = np.array([[1, 0], [-1, 0], [0, 1], [0, -1]], dtype=np.int32)


def _wid():
    return lax.axis_index("s") * 2 + lax.axis_index("c")


HBINS = 65536
SCAP = 1024
ELT = N // NW
HCH = 8192


def _build_kernel_h():
    mesh = plsc.VectorSubcoreMesh(core_axis_name="c", subcore_axis_name="s")
    NCH = ELT // HCH
    scratch = [
        pltpu.VMEM((HBINS,), jnp.int32),
        pltpu.VMEM((HCH,), jnp.int32),
        pltpu.VMEM((HCH,), jnp.int32),
        pltpu.SemaphoreType.DMA,
        pltpu.SemaphoreType.DMA,
    ]

    @functools.partial(
        pl.kernel, out_type=jax.ShapeDtypeStruct((NW * HBINS,), jnp.int32),
        mesh=mesh, scratch_types=scratch,
        compiler_params=pltpu.CompilerParams(needs_layout_passes=False))
    def kh(ki_hbm, z_hbm, hist_hbm, hist_v, ch0, ch1, sem0, sem1):
        wid = _wid()
        one16 = jnp.full((16,), 1, jnp.int32)
        pltpu.sync_copy(z_hbm, hist_v)
        base = wid * ELT
        bufs = (ch0, ch1)
        sems = (sem0, sem1)
        cps = {0: pltpu.async_copy(ki_hbm.at[pl.ds(base, HCH)], ch0, sem0)}
        for c in range(NCH):
            if c + 1 < NCH:
                cps[c + 1] = pltpu.async_copy(
                    ki_hbm.at[pl.ds(base + (c + 1) * HCH, HCH)],
                    bufs[(c + 1) % 2], sems[(c + 1) % 2])
            cps[c].wait()
            buf = bufs[c % 2]

            def hb(g, carry):
                g128 = g * 128
                for j in range(8):
                    k = buf[pl.ds(g128 + j * 16, 16)]
                    b = (k >> 16) + 32768
                    plsc.addupdate_scatter(hist_v, [b], one16)
                return carry
            lax.fori_loop(0, HCH // 128, hb, jnp.int32(0))
        pltpu.sync_copy(hist_v, hist_hbm.at[pl.ds(wid * HBINS, HBINS)])

    return kh


NSTR = 8
REG = 128
REGP = REG + 16


def _build_kernel_s():
    mesh = plsc.VectorSubcoreMesh(core_axis_name="c", subcore_axis_name="s")
    NCH = ELT // HCH
    out_type = [
        jax.ShapeDtypeStruct((NW * NSTR * REGP,), jnp.int32),
        jax.ShapeDtypeStruct((NW * NSTR * REGP,), jnp.int32),
        jax.ShapeDtypeStruct((NW * 16,), jnp.int32),
    ]
    scratch = [
        pltpu.VMEM((HCH,), jnp.int32),
        pltpu.VMEM((HCH,), jnp.int32),
        pltpu.VMEM((NSTR * REGP,), jnp.int32),
        pltpu.VMEM((NSTR * REGP,), jnp.int32),
        pltpu.VMEM((16,), jnp.int32),
        pltpu.SemaphoreType.DMA,
        pltpu.SemaphoreType.DMA,
    ]

    @functools.partial(
        pl.kernel, out_type=out_type, mesh=mesh, scratch_types=scratch,
        compiler_params=pltpu.CompilerParams(needs_layout_passes=False))
    def ks(ki_hbm, thr_hbm, sk_hbm, si_hbm, cnt_hbm,
           ch0, ch1, sk_v, si_v, thr_v, sem0, sem1):
        wid = _wid()
        pltpu.sync_copy(thr_hbm, thr_v)
        thr = thr_v[...]
        iot = lax.iota(jnp.int32, 16)
        one16 = jnp.full((16,), 1, jnp.int32)
        zero16 = jnp.zeros((16,), jnp.int32)
        base = wid * ELT
        bufs = (ch0, ch1)
        sems = (sem0, sem1)
        cps = {0: pltpu.async_copy(ki_hbm.at[pl.ds(base, HCH)], ch0, sem0)}
        offs = (jnp.int32(0),) * NSTR
        for c in range(NCH):
            if c + 1 < NCH:
                cps[c + 1] = pltpu.async_copy(
                    ki_hbm.at[pl.ds(base + (c + 1) * HCH, HCH)],
                    bufs[(c + 1) % 2], sems[(c + 1) % 2])
            cps[c].wait()
            buf = bufs[c % 2]
            cbase = base + c * HCH

            def sb(g, offs):
                g64 = g * (NSTR * 16)
                new = []
                for j in range(NSTR):
                    k = buf[pl.ds(g64 + j * 16, 16)]
                    m = k >= thr
                    gi = (cbase + g64 + j * 16) + iot
                    so = jnp.minimum(offs[j], REG)
                    plsc.store_compressed(
                        sk_v.at[pl.ds(j * REGP + so, 16)], k, mask=m)
                    plsc.store_compressed(
                        si_v.at[pl.ds(j * REGP + so, 16)], gi, mask=m)
                    new.append(offs[j] + jnp.sum(jnp.where(m, one16, zero16)))
                return tuple(new)
            offs = lax.fori_loop(0, HCH // (NSTR * 16), sb, offs)
        ob = wid * NSTR * REGP
        pltpu.sync_copy(sk_v, sk_hbm.at[pl.ds(ob, NSTR * REGP)])
        pltpu.sync_copy(si_v, si_hbm.at[pl.ds(ob, NSTR * REGP)])
        cvec = zero16
        for j in range(NSTR):
            cvec = jnp.where(iot == j, zero16 + jnp.minimum(offs[j], REG), cvec)
        thr_v[...] = cvec
        pltpu.sync_copy(thr_v, cnt_hbm.at[pl.ds(wid * 16, 16)])

    return ks


def _choice_topk(use_key, p_flat):
    g = jax.random.gumbel(use_key, (N,), dtype=jnp.float32) + jnp.log(p_flat)
    gb = lax.bitcast_convert_type(g, jnp.uint32)
    sgn = (gb >> 31).astype(bool)
    ku = gb ^ jnp.where(sgn, jnp.uint32(0xFFFFFFFF), jnp.uint32(0x80000000))
    ki = lax.bitcast_convert_type(ku ^ jnp.uint32(0x80000000), jnp.int32)

    hist = _build_kernel_h()(ki, jnp.zeros((HBINS,), jnp.int32)).reshape(NW, HBINS)
    tot = jnp.sum(hist, axis=0)
    rc = jnp.cumsum(tot[::-1])[::-1]
    bstar = jnp.max(jnp.where(rc >= NFA, jnp.arange(HBINS, dtype=jnp.int32), -1))
    thr = (bstar - 32768) * 65536
    thr16 = jnp.zeros((16,), jnp.int32) + thr

    sk, si, cnt = _build_kernel_s()(ki, thr16)
    counts = cnt.reshape(NW, 16)[:, :NSTR].reshape(-1)
    valid = (jnp.arange(REGP, dtype=jnp.int32)[None, :] < counts[:, None]).reshape(-1)
    ku_s = lax.bitcast_convert_type(sk, jnp.uint32) ^ jnp.uint32(0x80000000)
    sort1 = jnp.where(valid, ~ku_s, jnp.uint32(0xFFFFFFFF))
    idx2 = jnp.where(valid, si, N)
    _, s2 = lax.sort((sort1, idx2), num_keys=2)
    return s2[:NFA]


def _build_kernel_a():
    mesh = plsc.VectorSubcoreMesh(core_axis_name="c", subcore_axis_name="s")
    out_type = [jax.ShapeDtypeStruct((NFA,), jnp.float32) for _ in range(5)]
    scratch = [
        pltpu.VMEM((12 * APT,), jnp.int32),
        pltpu.VMEM((12 * APT,), jnp.float32),
        pltpu.VMEM((16,), jnp.float32),
        pltpu.VMEM((APT,), jnp.float32),
        pltpu.VMEM((APT,), jnp.float32),
        pltpu.VMEM((APT,), jnp.float32),
        pltpu.VMEM((APT,), jnp.float32),
        pltpu.VMEM((APT,), jnp.float32),
        pltpu.SemaphoreType.DMA,
    ]

    @functools.partial(pl.kernel, out_type=out_type, mesh=mesh,
                       scratch_types=scratch)
    def ka(gidx_hbm, cpm_hbm, rinv_hbm,
           d_hbm, vs0_hbm, vn0_hbm, vs1_hbm, vn1_hbm,
           idx_v, val_v, rinv_v, d_v, vs0_v, vn0_v, vs1_v, vn1_v, sem):
        wid = _wid()
        base = wid * (12 * APT)
        pltpu.sync_copy(gidx_hbm.at[pl.ds(base, 12 * APT)], idx_v)
        pltpu.sync_copy(rinv_hbm, rinv_v)
        cps = []
        for c in range(12 * APT // 128):
            cps.append(pltpu.async_copy(
                cpm_hbm.at[idx_v.at[pl.ds(c * 128, 128)]],
                val_v.at[pl.ds(c * 128, 128)], sem))
        for cp in cps:
            cp.wait()
        rv = rinv_v[...]
        one = jnp.full((16,), 1.0, jnp.float32)
        zero = jnp.zeros((16,), jnp.float32)
        for i in range(APT // 16):
            s = i * 16
            vsid = val_v[pl.ds(0 * APT + s, 16)]
            vnid = val_v[pl.ds(1 * APT + s, 16)]
            vsty = val_v[pl.ds(6 * APT + s, 16)]
            vnty = val_v[pl.ds(7 * APT + s, 16)]
            d = jnp.zeros((16,), jnp.float32)
            for k in range(4):
                nid = val_v[pl.ds((2 + k) * APT + s, 16)]
                nty = val_v[pl.ds((8 + k) * APT + s, 16)]
                d = d + (jnp.where(nid != vsid, one, zero)
                         - jnp.where(nid != vnid, one, zero))
                du = nty - vsty
                dv = nty - vnty
                d = d + 0.5 * (du * du - dv * dv)
            d_v[pl.ds(s, 16)] = rv * d
            vs0_v[pl.ds(s, 16)] = vsid
            vn0_v[pl.ds(s, 16)] = vnid
            vs1_v[pl.ds(s, 16)] = vsty
            vn1_v[pl.ds(s, 16)] = vnty
        ob = wid * APT
        pltpu.sync_copy(d_v, d_hbm.at[pl.ds(ob, APT)])
        pltpu.sync_copy(vs0_v, vs0_hbm.at[pl.ds(ob, APT)])
        pltpu.sync_copy(vn0_v, vn0_hbm.at[pl.ds(ob, APT)])
        pltpu.sync_copy(vs1_v, vs1_hbm.at[pl.ds(ob, APT)])
        pltpu.sync_copy(vn1_v, vn1_hbm.at[pl.ds(ob, APT)])

    return ka


def _build_kernel_b():
    mesh = plsc.VectorSubcoreMesh(core_axis_name="c", subcore_axis_name="s")
    CH = 2048
    CAP = NFA + 32
    PR = 8
    scratch = [
        pltpu.VMEM((CH,), jnp.int32),
        pltpu.VMEM((CH,), jnp.int32),
        pltpu.VMEM((CH,), jnp.int32),
        pltpu.VMEM((CH,), jnp.float32),
        pltpu.VMEM((CH,), jnp.float32),
        pltpu.VMEM((CH,), jnp.float32),
        pltpu.VMEM((CH,), jnp.float32),
        pltpu.VMEM((CAP,), jnp.int32),
        pltpu.VMEM((CAP,), jnp.float32),
        pltpu.VMEM((CAP,), jnp.float32),
        pltpu.VMEM((CAP,), jnp.int32),
        pltpu.VMEM((PR, L), jnp.float32),
        pltpu.VMEM((PR, L), jnp.float32),
        pltpu.VMEM((PR, L), jnp.float32),
        pltpu.VMEM((PR, L), jnp.float32),
        pltpu.SemaphoreType.DMA,
        pltpu.SemaphoreType.DMA,
        pltpu.SemaphoreType.DMA,
        pltpu.SemaphoreType.DMA,
    ]

    @functools.partial(
        pl.kernel, out_type=jax.ShapeDtypeStruct((2, L, L), jnp.float32),
        mesh=mesh, scratch_types=scratch,
        compiler_params=pltpu.CompilerParams(needs_layout_passes=False))
    def kb(cpm_hbm, row_hbm, col_hbm, acc_hbm,
           vs0_hbm, vn0_hbm, vs1_hbm, vn1_hbm, new_hbm,
           r_v, c_v, a_v, s0_v, n0_v, s1_v, n1_v,
           li_v, v0_v, v1_v, kp_v, bufa0, bufa1, bufb0, bufb1,
           isem0, isem1, osem0, osem1):
        wid = _wid()
        lo = wid * ROWS

        def ms(g, carry):
            li_v[pl.ds(g * 16, 16)] = jnp.full((16,), -1, jnp.int32)
            return carry
        lax.fori_loop(0, CAP // 16, ms, jnp.int32(0))

        off = jnp.int32(0)
        for ch in range(NFA // CH):
            cb = ch * CH
            pltpu.sync_copy(row_hbm.at[pl.ds(cb, CH)], r_v)
            pltpu.sync_copy(col_hbm.at[pl.ds(cb, CH)], c_v)
            pltpu.sync_copy(acc_hbm.at[pl.ds(cb, CH)], a_v)
            pltpu.sync_copy(vs0_hbm.at[pl.ds(cb, CH)], s0_v)
            pltpu.sync_copy(vn0_hbm.at[pl.ds(cb, CH)], n0_v)
            pltpu.sync_copy(vs1_hbm.at[pl.ds(cb, CH)], s1_v)
            pltpu.sync_copy(vn1_hbm.at[pl.ds(cb, CH)], n1_v)

            def fb(g, off):
                gs = g * 16
                rr = r_v[pl.ds(gs, 16)]
                cc = c_v[pl.ds(gs, 16)]
                aa = a_v[pl.ds(gs, 16)] != 0
                w0 = jnp.where(aa, s0_v[pl.ds(gs, 16)], n0_v[pl.ds(gs, 16)])
                w1 = jnp.where(aa, s1_v[pl.ds(gs, 16)], n1_v[pl.ds(gs, 16)])
                rel = rr - lo
                m = (rel >= 0) & (rel < ROWS)
                lidx = rel * L + cc
                plsc.store_compressed(li_v.at[pl.ds(off, 16)], lidx, mask=m)
                plsc.store_compressed(v0_v.at[pl.ds(off, 16)], w0, mask=m)
                plsc.store_compressed(v1_v.at[pl.ds(off, 16)], w1, mask=m)
                mi = jnp.where(m, jnp.full((16,), 1, jnp.int32),
                               jnp.zeros((16,), jnp.int32))
                return off + jnp.sum(mi)
            off = lax.fori_loop(0, CH // 16, fb, off)

        ngr = lax.div(off + 15, jnp.int32(16))

        def dd(j, carry):
            gs = j * 16
            lid = li_v[pl.ds(gs, 16)]
            keep = lid == li_v[pl.ds(gs, 16)]
            for sft in range(1, 16):
                keep = keep & (lid != li_v[pl.ds(gs + sft, 16)])
            kp_v[pl.ds(gs, 16)] = jnp.where(
                keep, jnp.full((16,), 1, jnp.int32),
                jnp.zeros((16,), jnp.int32))
            return carry
        lax.fori_loop(0, ngr, dd, jnp.int32(0))

        bufs = ((bufa0, bufa1), (bufb0, bufb1))
        isems = (isem0, isem1)
        osems = (osem0, osem1)
        NP = ROWS // PR
        in_cp = {}
        out_cp = {}

        def issue_in(p):
            par = p % 2
            r0 = lo + p * PR
            in_cp[p] = [
                pltpu.async_copy(cpm_hbm.at[0, pl.ds(r0, PR), :],
                                 bufs[par][0], isems[par]),
                pltpu.async_copy(cpm_hbm.at[1, pl.ds(r0, PR), :],
                                 bufs[par][1], isems[par]),
            ]

        issue_in(0)
        for p in range(NP):
            par = p % 2
            if p >= 1:
                for cp in out_cp[p - 1]:
                    cp.wait()
            if p + 1 < NP:
                issue_in(p + 1)
            for cp in in_cp[p]:
                cp.wait()
            lol = p * PR * L

            def pb(j, carry):
                gs = j * 16
                lid = li_v[pl.ds(gs, 16)]
                keep = kp_v[pl.ds(gs, 16)] != 0
                rel2 = lid - lol
                mm = (rel2 >= 0) & (rel2 < PR * L) & keep
                pidx = rel2 & (PR * L - 1)
                ri = pidx >> 11
                ci = pidx & (L - 1)
                plsc.store_scatter(bufs[par][0], [ri, ci],
                                   v0_v[pl.ds(gs, 16)], mask=mm)
                plsc.store_scatter(bufs[par][1], [ri, ci],
                                   v1_v[pl.ds(gs, 16)], mask=mm)
                return carry
            lax.fori_loop(0, ngr, pb, jnp.int32(0))
            r0 = lo + p * PR
            out_cp[p] = [
                pltpu.async_copy(bufs[par][0], new_hbm.at[0, pl.ds(r0, PR), :],
                                 osems[par]),
                pltpu.async_copy(bufs[par][1], new_hbm.at[1, pl.ds(r0, PR), :],
                                 osems[par]),
            ]
        for cp in out_cp[NP - 1]:
            cp.wait()

    return kb


def _kernel_c_call(cpm_new, accr):
    def body(cpm_ref, up_ref, dn_ref, acc_ref, mask_ref, e2_ref, as_ref):
        i = pl.program_id(0)
        ids = cpm_ref[0]
        tys = cpm_ref[1]
        uids = jnp.concatenate([up_ref[0, 7:8, :], ids[:-1]], axis=0)
        dids = jnp.concatenate([ids[1:], dn_ref[0, 0:1, :]], axis=0)
        lids = jnp.concatenate([ids[:, -1:], ids[:, :-1]], axis=1)
        rids = jnp.concatenate([ids[:, 1:], ids[:, :1]], axis=1)
        ne_u = ids != uids
        ne_l = ids != lids
        m = ne_u | (ids != dids) | ne_l | (ids != rids)
        mask_ref[...] = m.astype(jnp.float32)
        utys = jnp.concatenate([up_ref[1, 7:8, :], tys[:-1]], axis=0)
        ltys = jnp.concatenate([tys[:, -1:], tys[:, :-1]], axis=1)
        du = tys - utys
        dl = tys - ltys
        e2 = (2 * (ne_u.astype(jnp.int32) + ne_l.astype(jnp.int32))
              + (du * du).astype(jnp.int32) + (dl * dl).astype(jnp.int32))
        part = jnp.sum(e2)

        @pl.when(i == 0)
        def _():
            e2_ref[0, 0] = part
            as_ref[0, 0] = jnp.sum(acc_ref[...])

        @pl.when(i > 0)
        def _():
            e2_ref[0, 0] = e2_ref[0, 0] + part

    return pl.pallas_call(
        body,
        grid=(NW,),
        in_specs=[
            pl.BlockSpec((2, ROWS, L), lambda i: (0, i, 0)),
            pl.BlockSpec((2, 8, L), lambda i: (0, (i * 8 + L // 8 - 1) % (L // 8), 0)),
            pl.BlockSpec((2, 8, L), lambda i: (0, ((i + 1) % NW) * (ROWS // 8), 0)),
            pl.BlockSpec((64, 128), lambda i: (0, 0)),
        ],
        out_specs=[
            pl.BlockSpec((ROWS, L), lambda i: (i, 0)),
            pl.BlockSpec((1, 1), lambda i: (0, 0), memory_space=pltpu.SMEM),
            pl.BlockSpec((1, 1), lambda i: (0, 0), memory_space=pltpu.SMEM),
        ],
        out_shape=[
            jax.ShapeDtypeStruct((L, L), jnp.float32),
            jax.ShapeDtypeStruct((1, 1), jnp.int32),
            jax.ShapeDtypeStruct((1, 1), jnp.float32),
        ],
    )(cpm_new, cpm_new, cpm_new, accr)


def kernel(cpm, original_energy, boundary_mask, temperature, rng):
    key = rng
    key, use_key = jax.random.split(key)
    p = boundary_mask / boundary_mask.sum()
    p_flat = p.ravel()
    idx = _choice_topk(use_key, p_flat)
    sx, sy = jnp.unravel_index(idx, p.shape)
    key, key_ns = jax.random.split(key)
    keys_ns = jax.random.split(key_ns, NFA)

    def _draw(k):
        i = jax.random.randint(k, (), 0, 4)
        d = jnp.asarray(_NBR)[i]
        return d[0], d[1]

    dx, dy = jax.vmap(_draw)(keys_ns)
    nx = jnp.mod(sx + dx, L)
    ny = jnp.mod(sy + dy, L)
    key, use_key2 = jax.random.split(key)
    u = jax.random.uniform(use_key2, shape=(NFA,), minval=0.0, maxval=1.0)

    roles = [sx * L + sy, nx * L + ny]
    for k in range(4):
        ax = jnp.mod(nx + _NBR[k, 0], L)
        ay = jnp.mod(ny + _NBR[k, 1], L)
        roles.append(ax * L + ay)
    g0 = jnp.stack(roles).astype(jnp.int32)
    gidx = jnp.concatenate([g0, g0 + N], axis=0)
    gidx_t = gidx.reshape(12, NW, APT).transpose(1, 0, 2).reshape(-1)

    cpm_flat = cpm.reshape(2 * N)
    rinv = jnp.float32(1.0) / temperature
    rinv16 = jnp.full((16,), rinv, jnp.float32)

    deltas, vs0, vn0, vs1, vn1 = _build_kernel_a()(gidx_t, cpm_flat, rinv16)

    accepts = (u < jnp.exp(-deltas)).astype(jnp.int32)
    accepts_f = accepts.astype(jnp.float32)

    cpm_new = _build_kernel_b()(cpm, nx.astype(jnp.int32),
                                ny.astype(jnp.int32), accepts,
                                vs0, vn0, vs1, vn1)

    mask_new, e2, asum = _kernel_c_call(cpm_new,
                                        accepts_f.reshape(64, 128))
    energy = rinv * (e2[0, 0].astype(jnp.float32) * 0.5)
    delta_true = energy - original_energy
    accepts_sum = asum[0, 0]
    return (cpm_new, energy, mask_new, deltas, accepts_f, delta_true,
            accepts_sum)


# speedup vs baseline: 1.3149x; 1.3149x over previous
import functools

import numpy as np

import jax
import jax.numpy as jnp
from jax import lax
from jax.experimental import pallas as pl
from jax.experimental.pallas import tpu as pltpu
from jax.experimental.pallas import tpu_sc as plsc

L = 2048
N = L * L
NFA = 8192
NW = 32
APT = NFA // NW
ROWS = L // NW
_NBR = np.array([[1, 0], [-1, 0], [0, 1], [0, -1]], dtype=np.int32)


def _wid():
    return lax.axis_index("s") * 2 + lax.axis_index("c")


SCAP = 1024
ELT = N // NW


NSTR = 4
REG = 256
REGP = REG + 16


CROWS = 8


def _build_kernel_s():
    mesh = plsc.VectorSubcoreMesh(core_axis_name="c", subcore_axis_name="s")
    NCH = ROWS // CROWS
    out_type = [
        jax.ShapeDtypeStruct((NW * NSTR * REGP,), jnp.int32),
        jax.ShapeDtypeStruct((NW * NSTR * REGP,), jnp.int32),
        jax.ShapeDtypeStruct((NW * 16,), jnp.int32),
    ]
    scratch = [
        pltpu.VMEM((CROWS, L), jnp.int32),
        pltpu.VMEM((CROWS, L), jnp.int32),
        pltpu.VMEM((NSTR * REGP,), jnp.int32),
        pltpu.VMEM((NSTR * REGP,), jnp.int32),
        pltpu.VMEM((16,), jnp.int32),
        pltpu.SemaphoreType.DMA,
        pltpu.SemaphoreType.DMA,
    ]

    @functools.partial(
        pl.kernel, out_type=out_type, mesh=mesh, scratch_types=scratch,
        compiler_params=pltpu.CompilerParams(needs_layout_passes=False))
    def ks(ki_hbm, thr_hbm, sk_hbm, si_hbm, cnt_hbm,
           ch0, ch1, sk_v, si_v, thr_v, sem0, sem1):
        wid = _wid()
        pltpu.sync_copy(thr_hbm, thr_v)
        thr = thr_v[...]
        iot = lax.iota(jnp.int32, 16)
        one16 = jnp.full((16,), 1, jnp.int32)
        zero16 = jnp.zeros((16,), jnp.int32)
        row0 = wid * ROWS
        bufs = (ch0, ch1)
        sems = (sem0, sem1)
        cps = {0: pltpu.async_copy(ki_hbm.at[pl.ds(row0, CROWS), :], ch0, sem0)}
        offs = (jnp.int32(0),) * NSTR
        for c in range(NCH):
            if c + 1 < NCH:
                cps[c + 1] = pltpu.async_copy(
                    ki_hbm.at[pl.ds(row0 + (c + 1) * CROWS, CROWS), :],
                    bufs[(c + 1) % 2], sems[(c + 1) % 2])
            cps[c].wait()
            buf = bufs[c % 2]
            for r in range(CROWS):
                rbase = (row0 + c * CROWS + r) * L

                def sb(g, offs):
                    cb = g * (NSTR * 16)
                    new = []
                    for j in range(NSTR):
                        col = cb + j * 16
                        k = buf[r, pl.ds(col, 16)]
                        m = k >= thr
                        gi = (rbase + col) + iot
                        so = jnp.minimum(offs[j], REG)
                        plsc.store_compressed(
                            sk_v.at[pl.ds(j * REGP + so, 16)], k, mask=m)
                        plsc.store_compressed(
                            si_v.at[pl.ds(j * REGP + so, 16)], gi, mask=m)
                        new.append(offs[j] + jnp.sum(jnp.where(m, one16, zero16)))
                    return tuple(new)
                offs = lax.fori_loop(0, L // (NSTR * 16), sb, offs)
        ob = wid * NSTR * REGP
        pltpu.sync_copy(sk_v, sk_hbm.at[pl.ds(ob, NSTR * REGP)])
        pltpu.sync_copy(si_v, si_hbm.at[pl.ds(ob, NSTR * REGP)])
        cvec = zero16
        for j in range(NSTR):
            cvec = jnp.where(iot == j, zero16 + jnp.minimum(offs[j], REG), cvec)
        thr_v[...] = cvec
        pltpu.sync_copy(thr_v, cnt_hbm.at[pl.ds(wid * 16, 16)])

    return ks


def _key_i32(x):
    b = lax.bitcast_convert_type(x, jnp.uint32)
    sgn = (b >> 31).astype(bool)
    ku = b ^ jnp.where(sgn, jnp.uint32(0xFFFFFFFF), jnp.uint32(0x80000000))
    return lax.bitcast_convert_type(ku ^ jnp.uint32(0x80000000), jnp.int32)


def _choice_topk(use_key, p2d, bsum):
    g = jax.random.gumbel(use_key, (L, L), dtype=jnp.float32) + jnp.log(p2d)
    ki = _key_i32(g)

    q = jnp.minimum(jnp.float32(12288.0) / bsum, jnp.float32(0.2))
    x = -jnp.log(-jnp.log1p(-q))
    tg = x - jnp.log(bsum)
    thr16 = jnp.zeros((16,), jnp.int32) + _key_i32(tg.astype(jnp.float32))

    sk, si, cnt = _build_kernel_s()(ki, thr16)
    counts = cnt.reshape(NW, 16)[:, :NSTR].reshape(-1)
    valid = (jnp.arange(REGP, dtype=jnp.int32)[None, :] < counts[:, None]).reshape(-1)
    ku_s = lax.bitcast_convert_type(sk, jnp.uint32) ^ jnp.uint32(0x80000000)
    sort1 = jnp.where(valid, ~ku_s, jnp.uint32(0xFFFFFFFF))
    idx2 = jnp.where(valid, si, N)
    _, s2 = lax.sort((sort1, idx2), num_keys=2)
    return s2[:NFA]


def _build_kernel_a():
    mesh = plsc.VectorSubcoreMesh(core_axis_name="c", subcore_axis_name="s")
    out_type = [jax.ShapeDtypeStruct((NFA,), jnp.float32) for _ in range(5)]
    scratch = [
        pltpu.VMEM((12 * APT,), jnp.int32),
        pltpu.VMEM((12 * APT,), jnp.float32),
        pltpu.VMEM((16,), jnp.float32),
        pltpu.VMEM((APT,), jnp.float32),
        pltpu.VMEM((APT,), jnp.float32),
        pltpu.VMEM((APT,), jnp.float32),
        pltpu.VMEM((APT,), jnp.float32),
        pltpu.VMEM((APT,), jnp.float32),
        pltpu.SemaphoreType.DMA,
    ]

    @functools.partial(pl.kernel, out_type=out_type, mesh=mesh,
                       scratch_types=scratch)
    def ka(gidx_hbm, cpm_hbm, rinv_hbm,
           d_hbm, vs0_hbm, vn0_hbm, vs1_hbm, vn1_hbm,
           idx_v, val_v, rinv_v, d_v, vs0_v, vn0_v, vs1_v, vn1_v, sem):
        wid = _wid()
        base = wid * (12 * APT)
        pltpu.sync_copy(gidx_hbm.at[pl.ds(base, 12 * APT)], idx_v)
        pltpu.sync_copy(rinv_hbm, rinv_v)
        cps = []
        for c in range(12 * APT // 128):
            cps.append(pltpu.async_copy(
                cpm_hbm.at[idx_v.at[pl.ds(c * 128, 128)]],
                val_v.at[pl.ds(c * 128, 128)], sem))
        for cp in cps:
            cp.wait()
        rv = rinv_v[...]
        one = jnp.full((16,), 1.0, jnp.float32)
        zero = jnp.zeros((16,), jnp.float32)
        for i in range(APT // 16):
            s = i * 16
            vsid = val_v[pl.ds(0 * APT + s, 16)]
            vnid = val_v[pl.ds(1 * APT + s, 16)]
            vsty = val_v[pl.ds(6 * APT + s, 16)]
            vnty = val_v[pl.ds(7 * APT + s, 16)]
            d = jnp.zeros((16,), jnp.float32)
            for k in range(4):
                nid = val_v[pl.ds((2 + k) * APT + s, 16)]
                nty = val_v[pl.ds((8 + k) * APT + s, 16)]
                d = d + (jnp.where(nid != vsid, one, zero)
                         - jnp.where(nid != vnid, one, zero))
                du = nty - vsty
                dv = nty - vnty
                d = d + 0.5 * (du * du - dv * dv)
            d_v[pl.ds(s, 16)] = rv * d
            vs0_v[pl.ds(s, 16)] = vsid
            vn0_v[pl.ds(s, 16)] = vnid
            vs1_v[pl.ds(s, 16)] = vsty
            vn1_v[pl.ds(s, 16)] = vnty
        ob = wid * APT
        pltpu.sync_copy(d_v, d_hbm.at[pl.ds(ob, APT)])
        pltpu.sync_copy(vs0_v, vs0_hbm.at[pl.ds(ob, APT)])
        pltpu.sync_copy(vn0_v, vn0_hbm.at[pl.ds(ob, APT)])
        pltpu.sync_copy(vs1_v, vs1_hbm.at[pl.ds(ob, APT)])
        pltpu.sync_copy(vn1_v, vn1_hbm.at[pl.ds(ob, APT)])

    return ka


def _build_kernel_b():
    mesh = plsc.VectorSubcoreMesh(core_axis_name="c", subcore_axis_name="s")
    CH = 2048
    CAP = NFA + 32
    PR = 8
    scratch = [
        pltpu.VMEM((CH,), jnp.int32),
        pltpu.VMEM((CH,), jnp.int32),
        pltpu.VMEM((CH,), jnp.int32),
        pltpu.VMEM((CH,), jnp.float32),
        pltpu.VMEM((CH,), jnp.float32),
        pltpu.VMEM((CH,), jnp.float32),
        pltpu.VMEM((CH,), jnp.float32),
        pltpu.VMEM((CAP,), jnp.int32),
        pltpu.VMEM((CAP,), jnp.float32),
        pltpu.VMEM((CAP,), jnp.float32),
        pltpu.VMEM((CAP,), jnp.int32),
        pltpu.VMEM((PR, L), jnp.float32),
        pltpu.VMEM((PR, L), jnp.float32),
        pltpu.VMEM((PR, L), jnp.float32),
        pltpu.VMEM((PR, L), jnp.float32),
        pltpu.SemaphoreType.DMA,
        pltpu.SemaphoreType.DMA,
        pltpu.SemaphoreType.DMA,
        pltpu.SemaphoreType.DMA,
    ]

    @functools.partial(
        pl.kernel, out_type=jax.ShapeDtypeStruct((2, L, L), jnp.float32),
        mesh=mesh, scratch_types=scratch,
        compiler_params=pltpu.CompilerParams(needs_layout_passes=False))
    def kb(cpm_hbm, row_hbm, col_hbm, acc_hbm,
           vs0_hbm, vn0_hbm, vs1_hbm, vn1_hbm, new_hbm,
           r_v, c_v, a_v, s0_v, n0_v, s1_v, n1_v,
           li_v, v0_v, v1_v, kp_v, bufa0, bufa1, bufb0, bufb1,
           isem0, isem1, osem0, osem1):
        wid = _wid()
        lo = wid * ROWS

        def ms(g, carry):
            li_v[pl.ds(g * 16, 16)] = jnp.full((16,), -1, jnp.int32)
            return carry
        lax.fori_loop(0, CAP // 16, ms, jnp.int32(0))

        off = jnp.int32(0)
        for ch in range(NFA // CH):
            cb = ch * CH
            pltpu.sync_copy(row_hbm.at[pl.ds(cb, CH)], r_v)
            pltpu.sync_copy(col_hbm.at[pl.ds(cb, CH)], c_v)
            pltpu.sync_copy(acc_hbm.at[pl.ds(cb, CH)], a_v)
            pltpu.sync_copy(vs0_hbm.at[pl.ds(cb, CH)], s0_v)
            pltpu.sync_copy(vn0_hbm.at[pl.ds(cb, CH)], n0_v)
            pltpu.sync_copy(vs1_hbm.at[pl.ds(cb, CH)], s1_v)
            pltpu.sync_copy(vn1_hbm.at[pl.ds(cb, CH)], n1_v)

            def fb(g, off):
                gs = g * 16
                rr = r_v[pl.ds(gs, 16)]
                cc = c_v[pl.ds(gs, 16)]
                aa = a_v[pl.ds(gs, 16)] != 0
                w0 = jnp.where(aa, s0_v[pl.ds(gs, 16)], n0_v[pl.ds(gs, 16)])
                w1 = jnp.where(aa, s1_v[pl.ds(gs, 16)], n1_v[pl.ds(gs, 16)])
                rel = rr - lo
                m = (rel >= 0) & (rel < ROWS)
                lidx = rel * L + cc
                plsc.store_compressed(li_v.at[pl.ds(off, 16)], lidx, mask=m)
                plsc.store_compressed(v0_v.at[pl.ds(off, 16)], w0, mask=m)
                plsc.store_compressed(v1_v.at[pl.ds(off, 16)], w1, mask=m)
                mi = jnp.where(m, jnp.full((16,), 1, jnp.int32),
                               jnp.zeros((16,), jnp.int32))
                return off + jnp.sum(mi)
            off = lax.fori_loop(0, CH // 16, fb, off)

        ngr = lax.div(off + 15, jnp.int32(16))

        def dd(j, carry):
            gs = j * 16
            lid = li_v[pl.ds(gs, 16)]
            keep = lid == li_v[pl.ds(gs, 16)]
            for sft in range(1, 16):
                keep = keep & (lid != li_v[pl.ds(gs + sft, 16)])
            kp_v[pl.ds(gs, 16)] = jnp.where(
                keep, jnp.full((16,), 1, jnp.int32),
                jnp.zeros((16,), jnp.int32))
            return carry
        lax.fori_loop(0, ngr, dd, jnp.int32(0))

        bufs = ((bufa0, bufa1), (bufb0, bufb1))
        isems = (isem0, isem1)
        osems = (osem0, osem1)
        NP = ROWS // PR
        in_cp = {}
        out_cp = {}

        def issue_in(p):
            par = p % 2
            r0 = lo + p * PR
            in_cp[p] = [
                pltpu.async_copy(cpm_hbm.at[0, pl.ds(r0, PR), :],
                                 bufs[par][0], isems[par]),
                pltpu.async_copy(cpm_hbm.at[1, pl.ds(r0, PR), :],
                                 bufs[par][1], isems[par]),
            ]

        issue_in(0)
        for p in range(NP):
            par = p % 2
            if p >= 1:
                for cp in out_cp[p - 1]:
                    cp.wait()
            if p + 1 < NP:
                issue_in(p + 1)
            for cp in in_cp[p]:
                cp.wait()
            lol = p * PR * L

            def pb(j, carry):
                gs = j * 16
                lid = li_v[pl.ds(gs, 16)]
                keep = kp_v[pl.ds(gs, 16)] != 0
                rel2 = lid - lol
                mm = (rel2 >= 0) & (rel2 < PR * L) & keep
                pidx = rel2 & (PR * L - 1)
                ri = pidx >> 11
                ci = pidx & (L - 1)
                plsc.store_scatter(bufs[par][0], [ri, ci],
                                   v0_v[pl.ds(gs, 16)], mask=mm)
                plsc.store_scatter(bufs[par][1], [ri, ci],
                                   v1_v[pl.ds(gs, 16)], mask=mm)
                return carry
            lax.fori_loop(0, ngr, pb, jnp.int32(0))
            r0 = lo + p * PR
            out_cp[p] = [
                pltpu.async_copy(bufs[par][0], new_hbm.at[0, pl.ds(r0, PR), :],
                                 osems[par]),
                pltpu.async_copy(bufs[par][1], new_hbm.at[1, pl.ds(r0, PR), :],
                                 osems[par]),
            ]
        for cp in out_cp[NP - 1]:
            cp.wait()

    return kb


def _kernel_c_call(cpm_new, accr):
    def body(cpm_ref, up_ref, dn_ref, acc_ref, mask_ref, e2_ref, as_ref):
        i = pl.program_id(0)
        ids = cpm_ref[0]
        tys = cpm_ref[1]
        uids = jnp.concatenate([up_ref[0, 7:8, :], ids[:-1]], axis=0)
        dids = jnp.concatenate([ids[1:], dn_ref[0, 0:1, :]], axis=0)
        lids = jnp.concatenate([ids[:, -1:], ids[:, :-1]], axis=1)
        rids = jnp.concatenate([ids[:, 1:], ids[:, :1]], axis=1)
        ne_u = ids != uids
        ne_l = ids != lids
        m = ne_u | (ids != dids) | ne_l | (ids != rids)
        mask_ref[...] = m.astype(jnp.float32)
        utys = jnp.concatenate([up_ref[1, 7:8, :], tys[:-1]], axis=0)
        ltys = jnp.concatenate([tys[:, -1:], tys[:, :-1]], axis=1)
        du = tys - utys
        dl = tys - ltys
        e2 = (2 * (ne_u.astype(jnp.int32) + ne_l.astype(jnp.int32))
              + (du * du).astype(jnp.int32) + (dl * dl).astype(jnp.int32))
        part = jnp.sum(e2)

        @pl.when(i == 0)
        def _():
            e2_ref[0, 0] = part
            as_ref[0, 0] = jnp.sum(acc_ref[...])

        @pl.when(i > 0)
        def _():
            e2_ref[0, 0] = e2_ref[0, 0] + part

    return pl.pallas_call(
        body,
        grid=(NW,),
        in_specs=[
            pl.BlockSpec((2, ROWS, L), lambda i: (0, i, 0)),
            pl.BlockSpec((2, 8, L), lambda i: (0, (i * 8 + L // 8 - 1) % (L // 8), 0)),
            pl.BlockSpec((2, 8, L), lambda i: (0, ((i + 1) % NW) * (ROWS // 8), 0)),
            pl.BlockSpec((64, 128), lambda i: (0, 0)),
        ],
        out_specs=[
            pl.BlockSpec((ROWS, L), lambda i: (i, 0)),
            pl.BlockSpec((1, 1), lambda i: (0, 0), memory_space=pltpu.SMEM),
            pl.BlockSpec((1, 1), lambda i: (0, 0), memory_space=pltpu.SMEM),
        ],
        out_shape=[
            jax.ShapeDtypeStruct((L, L), jnp.float32),
            jax.ShapeDtypeStruct((1, 1), jnp.int32),
            jax.ShapeDtypeStruct((1, 1), jnp.float32),
        ],
    )(cpm_new, cpm_new, cpm_new, accr)


def kernel(cpm, original_energy, boundary_mask, temperature, rng):
    key = rng
    key, use_key = jax.random.split(key)
    bsum = boundary_mask.sum()
    p = boundary_mask / bsum
    idx = _choice_topk(use_key, p, bsum)
    sx, sy = jnp.unravel_index(idx, p.shape)
    key, key_ns = jax.random.split(key)
    keys_ns = jax.random.split(key_ns, NFA)

    def _draw(k):
        i = jax.random.randint(k, (), 0, 4)
        d = jnp.asarray(_NBR)[i]
        return d[0], d[1]

    dx, dy = jax.vmap(_draw)(keys_ns)
    nx = jnp.mod(sx + dx, L)
    ny = jnp.mod(sy + dy, L)
    key, use_key2 = jax.random.split(key)
    u = jax.random.uniform(use_key2, shape=(NFA,), minval=0.0, maxval=1.0)

    roles = [sx * L + sy, nx * L + ny]
    for k in range(4):
        ax = jnp.mod(nx + _NBR[k, 0], L)
        ay = jnp.mod(ny + _NBR[k, 1], L)
        roles.append(ax * L + ay)
    g0 = jnp.stack(roles).astype(jnp.int32)
    gidx = jnp.concatenate([g0, g0 + N], axis=0)
    gidx_t = gidx.reshape(12, NW, APT).transpose(1, 0, 2).reshape(-1)

    cpm_flat = cpm.reshape(2 * N)
    rinv = jnp.float32(1.0) / temperature
    rinv16 = jnp.full((16,), rinv, jnp.float32)

    deltas, vs0, vn0, vs1, vn1 = _build_kernel_a()(gidx_t, cpm_flat, rinv16)

    accepts = (u < jnp.exp(-deltas)).astype(jnp.int32)
    accepts_f = accepts.astype(jnp.float32)

    cpm_new = _build_kernel_b()(cpm, nx.astype(jnp.int32),
                                ny.astype(jnp.int32), accepts,
                                vs0, vn0, vs1, vn1)

    mask_new, e2, asum = _kernel_c_call(cpm_new,
                                        accepts_f.reshape(64, 128))
    energy = rinv * (e2[0, 0].astype(jnp.float32) * 0.5)
    delta_true = energy - original_energy
    accepts_sum = asum[0, 0]
    return (cpm_new, energy, mask_new, deltas, accepts_f, delta_true,
            accepts_sum)
